# Initial kernel scaffold; baseline (speedup 1.0000x reference)
import jax
import jax.numpy as jnp
from jax.experimental import pallas as pl


def kernel(label_idx, embedding_center):
    raise NotImplementedError("write your pallas kernel here")


# speedup vs baseline: 1.1124x; 1.1124x over previous
import functools

import jax
import jax.numpy as jnp
from jax import lax
from jax.experimental import pallas as pl
from jax.experimental.pallas import tpu as pltpu
from jax.experimental.pallas import tpu_sc as plsc

VOCAB = 1_000_000
D = 32
BT = 16384 * 50
NC, NS = 2, 16
NW = NC * NS
PW = BT // NW
IW = 128
NROW = PW // IW
CH = 10
CHROWS = CH * IW
NB = 2
NOUT = NROW // (CH * NB)


@functools.cache
def _build():
    mesh = plsc.VectorSubcoreMesh(
        core_axis_name="c", subcore_axis_name="s",
        num_cores=NC, num_subcores=NS)

    @functools.partial(
        pl.kernel,
        out_type=jax.ShapeDtypeStruct((BT, D), jnp.float32),
        mesh=mesh,
        compiler_params=pltpu.CompilerParams(use_tc_tiling_on_sc=False),
        scratch_types=[
            pltpu.VMEM((NROW, IW), jnp.int32),
            pltpu.VMEM((NB, CHROWS, D), jnp.float32),
            pltpu.SemaphoreType.DMA,
            pltpu.SemaphoreType.DMA,
        ],
    )
    def k(idx_hbm, table_hbm, out_hbm, idx_v, rows, sem_g, sem_out):
        wid = lax.axis_index("s") * NC + lax.axis_index("c")
        pltpu.sync_copy(idx_hbm.at[wid], idx_v)
        out_base = wid * PW

        @pl.loop(0, NOUT)
        def body(g):
            @pl.when(g > 0)
            def _():
                for b in range(NB):
                    pltpu.make_async_copy(
                        rows.at[b], out_hbm.at[pl.ds(0, CHROWS)], sem_out
                    ).wait()

            descs = []
            for b in range(NB):
                c = g * NB + b
                for j in range(CH):
                    d = pltpu.async_copy(
                        table_hbm.at[idx_v.at[c * CH + j]],
                        rows.at[b, pl.ds(j * IW, IW)],
                        sem_g)
                    descs.append(d)
            for b in range(NB):
                for j in range(CH):
                    descs[b * CH + j].wait()
                c = g * NB + b
                pltpu.async_copy(
                    rows.at[b],
                    out_hbm.at[pl.ds(out_base + c * CHROWS, CHROWS)],
                    sem_out)

        for b in range(NB):
            pltpu.make_async_copy(
                rows.at[b], out_hbm.at[pl.ds(0, CHROWS)], sem_out
            ).wait()

    return k


def kernel(label_idx, embedding_center):
    idx3 = label_idx.astype(jnp.int32).reshape(NW, NROW, IW)
    out = _build()(idx3, embedding_center)
    return out.reshape(label_idx.shape + (D,))


# speedup vs baseline: 1.7886x; 1.6079x over previous
import functools

import jax
import jax.numpy as jnp
from jax import lax
from jax.experimental import pallas as pl
from jax.experimental.pallas import tpu as pltpu
from jax.experimental.pallas import tpu_sc as plsc

B = 16384
H = 50
D = 32
NC, NS = 2, 16
NW = NC * NS
PB = B // NW
K = 8
NB = 2
NOUT = PB // (K * NB)


@functools.cache
def _build():
    mesh = plsc.VectorSubcoreMesh(
        core_axis_name="c", subcore_axis_name="s",
        num_cores=NC, num_subcores=NS)

    @functools.partial(
        pl.kernel,
        out_type=jax.ShapeDtypeStruct((B, H, D), jnp.float32),
        mesh=mesh,
        compiler_params=pltpu.CompilerParams(use_tc_tiling_on_sc=False),
        scratch_types=[
            pltpu.VMEM((PB, H), jnp.int32),
            pltpu.VMEM((NB, K, H, D), jnp.float32),
            pltpu.SemaphoreType.DMA,
            pltpu.SemaphoreType.DMA,
        ],
    )
    def k(idx_hbm, table_hbm, out_hbm, idx_v, rows, sem_g, sem_out):
        wid = lax.axis_index("s") * NC + lax.axis_index("c")
        base = wid * PB
        pltpu.sync_copy(idx_hbm.at[pl.ds(base, PB)], idx_v)

        @pl.loop(0, NOUT)
        def body(g):
            @pl.when(g > 0)
            def _():
                for b in range(NB):
                    pltpu.make_async_copy(
                        rows.at[b], out_hbm.at[pl.ds(0, K)], sem_out
                    ).wait()

            descs = []
            for b in range(NB):
                c = g * NB + b
                for j in range(K):
                    d = pltpu.async_copy(
                        table_hbm.at[idx_v.at[c * K + j]],
                        rows.at[b, j],
                        sem_g)
                    descs.append(d)
            for b in range(NB):
                for j in range(K):
                    descs[b * K + j].wait()
                c = g * NB + b
                pltpu.async_copy(
                    rows.at[b],
                    out_hbm.at[pl.ds(base + c * K, K)],
                    sem_out)

        for b in range(NB):
            pltpu.make_async_copy(
                rows.at[b], out_hbm.at[pl.ds(0, K)], sem_out
            ).wait()

    return k


def kernel(label_idx, embedding_center):
    return _build()(label_idx.astype(jnp.int32), embedding_center)
